# baseline (speedup 1.0000x reference)
import jax
import jax.numpy as jnp
from jax.experimental import pallas as pl
from jax.experimental.pallas import tpu as pltpu

_T, _D, _F, _G, _E = 2048, 768, 3072, 3, 8
_GE = _G * _E
_FC = 1024
_NF = _F // _FC
_S = _GE * _NF


def _moe_body(x_ref, wg_ref, bg_ref, wr_ref, br_ref, b2r_ref,
              w1_ref, b1_ref, w2_ref, out_ref,
              w_scr, w1bf_scr, w2bf_scr, b1_scr):
    s = pl.program_id(0)

    @pl.when(s == 0)
    def _init():
        x = x_ref[...]
        gl = jnp.dot(x, wg_ref[...], preferred_element_type=jnp.float32)
        gl = gl + bg_ref[...]
        gl = gl - jnp.max(gl, axis=1, keepdims=True)
        gp = jnp.exp(gl)
        gp = gp / jnp.sum(gp, axis=1, keepdims=True)
        el = jnp.dot(x, wr_ref[...], preferred_element_type=jnp.float32)
        el = el + br_ref[...]
        cols = []
        for g in range(_G):
            sl = el[:, g * _E:(g + 1) * _E]
            sl = sl - jnp.max(sl, axis=1, keepdims=True)
            p = jnp.exp(sl)
            p = p / jnp.sum(p, axis=1, keepdims=True)
            cols.append(p * gp[:, g:g + 1])
        w = jnp.concatenate(cols, axis=1)
        w_scr[...] = w * 0.5
        out_ref[...] = jnp.dot(w, b2r_ref[...],
                               preferred_element_type=jnp.float32)

    @pl.when(s < _S)
    def _cast_stage():
        par = jax.lax.rem(s, 2)
        w1bf_scr[par] = w1_ref[0].astype(jnp.bfloat16)
        w2bf_scr[par] = w2_ref[0].astype(jnp.bfloat16)
        b1_scr[par] = b1_ref[0]

    @pl.when(s > 0)
    def _matmul_stage():
        j = s - 1
        par = jax.lax.rem(j, 2)
        e = j // _NF
        x = x_ref[...]
        t = jnp.dot(x, w1bf_scr[par],
                    preferred_element_type=jnp.float32)
        t = t + b1_scr[par]
        lane = jax.lax.broadcasted_iota(jnp.int32, (_T, _GE), 1)
        wselh = jnp.sum(jnp.where(lane == e, w_scr[...], 0.0),
                        axis=1, keepdims=True).astype(jnp.bfloat16)
        t_bf = t.astype(jnp.bfloat16)
        v = jax.lax.erf(t_bf * jnp.bfloat16(0.7071067811865476))
        a = t_bf * wselh
        h = a + a * v
        out_ref[...] += jnp.dot(h, w2bf_scr[par],
                                preferred_element_type=jnp.float32)


def kernel(h_fused, Wg, bg, Wr, br, W1, b1, W2, b2):
    x_bf = h_fused.astype(jnp.bfloat16)
    wg_bf = Wg.astype(jnp.bfloat16)
    wr_bf = Wr.transpose(1, 0, 2).reshape(_D, _GE).astype(jnp.bfloat16)
    bg2 = bg.reshape(1, _G)
    br2 = br.reshape(1, _GE)
    w1r = W1.reshape(_GE, _D, _F)
    b1r = b1.reshape(_GE, 1, _F)
    w2r = W2.reshape(_GE, _F, _D)
    b2r = b2.reshape(_GE, _D)

    def _w1_idx(s):
        c = jnp.minimum(s, _S - 1)
        return (c // _NF, 0, jax.lax.rem(c, _NF))

    def _b1_idx(s):
        c = jnp.minimum(s, _S - 1)
        return (c // _NF, 0, jax.lax.rem(c, _NF))

    def _w2_idx(s):
        c = jnp.minimum(s, _S - 1)
        return (c // _NF, jax.lax.rem(c, _NF), 0)

    out = pl.pallas_call(
        _moe_body,
        grid=(_S + 1,),
        in_specs=[
            pl.BlockSpec((_T, _D), lambda s: (0, 0)),
            pl.BlockSpec((_D, _G), lambda s: (0, 0)),
            pl.BlockSpec((1, _G), lambda s: (0, 0)),
            pl.BlockSpec((_D, _GE), lambda s: (0, 0)),
            pl.BlockSpec((1, _GE), lambda s: (0, 0)),
            pl.BlockSpec((_GE, _D), lambda s: (0, 0)),
            pl.BlockSpec((1, _D, _FC), _w1_idx),
            pl.BlockSpec((1, 1, _FC), _b1_idx),
            pl.BlockSpec((1, _FC, _D), _w2_idx),
        ],
        out_specs=pl.BlockSpec((_T, _D), lambda s: (0, 0)),
        out_shape=jax.ShapeDtypeStruct((_T, _D), jnp.float32),
        scratch_shapes=[pltpu.VMEM((_T, _GE), jnp.float32),
                        pltpu.VMEM((2, _D, _FC), jnp.bfloat16),
                        pltpu.VMEM((2, _FC, _D), jnp.bfloat16),
                        pltpu.VMEM((2, 1, _FC), jnp.float32)],
    )(x_bf, wg_bf, bg2, wr_bf, br2, b2r, w1r, b1r, w2r)
    return out


# speedup vs baseline: 1.0366x; 1.0366x over previous
import jax
import jax.numpy as jnp
from jax.experimental import pallas as pl
from jax.experimental.pallas import tpu as pltpu

_T, _D, _F, _G, _E = 2048, 768, 3072, 3, 8
_GE = _G * _E
_FC = 1024
_NF = _F // _FC


def _moe_body(x_ref, wg_ref, bg_ref, wr_ref, br_ref, b2r_ref,
              w1_ref, b1_ref, w2_ref, out_ref, w_scr):
    e = pl.program_id(0)
    f = pl.program_id(1)

    @pl.when((e == 0) & (f == 0))
    def _init():
        x = x_ref[...]
        gl = jnp.dot(x, wg_ref[...], preferred_element_type=jnp.float32)
        gl = gl + bg_ref[...]
        gl = gl - jnp.max(gl, axis=1, keepdims=True)
        gp = jnp.exp(gl)
        gp = gp / jnp.sum(gp, axis=1, keepdims=True)
        el = jnp.dot(x, wr_ref[...], preferred_element_type=jnp.float32)
        el = el + br_ref[...]
        cols = []
        for g in range(_G):
            sl = el[:, g * _E:(g + 1) * _E]
            sl = sl - jnp.max(sl, axis=1, keepdims=True)
            p = jnp.exp(sl)
            p = p / jnp.sum(p, axis=1, keepdims=True)
            cols.append(p * gp[:, g:g + 1])
        w = jnp.concatenate(cols, axis=1)
        w_scr[...] = w * 0.5
        out_ref[...] = jnp.dot(w, b2r_ref[...],
                               preferred_element_type=jnp.float32)

    x = x_ref[...]
    w1 = w1_ref[0].astype(jnp.bfloat16)
    t = jnp.dot(x, w1, preferred_element_type=jnp.float32)
    t = t + b1_ref[0]
    lane = jax.lax.broadcasted_iota(jnp.int32, (_T, _GE), 1)
    wselh = jnp.sum(jnp.where(lane == e, w_scr[...], 0.0),
                    axis=1, keepdims=True).astype(jnp.bfloat16)
    t_bf = t.astype(jnp.bfloat16)
    v = jax.lax.erf(t_bf * jnp.bfloat16(0.7071067811865476))
    a = t_bf * wselh
    h = a + a * v
    w2 = w2_ref[0].astype(jnp.bfloat16)
    out_ref[...] += jnp.dot(h, w2, preferred_element_type=jnp.float32)


def kernel(h_fused, Wg, bg, Wr, br, W1, b1, W2, b2):
    x_bf = h_fused.astype(jnp.bfloat16)
    wg_bf = Wg.astype(jnp.bfloat16)
    wr_bf = Wr.transpose(1, 0, 2).reshape(_D, _GE).astype(jnp.bfloat16)
    bg2 = bg.reshape(1, _G)
    br2 = br.reshape(1, _GE)
    w1r = W1.reshape(_GE, _D, _F)
    b1r = b1.reshape(_GE, 1, _F)
    w2r = W2.reshape(_GE, _F, _D)
    b2r = b2.reshape(_GE, _D)

    out = pl.pallas_call(
        _moe_body,
        grid=(_GE, _NF),
        in_specs=[
            pl.BlockSpec((_T, _D), lambda e, f: (0, 0)),
            pl.BlockSpec((_D, _G), lambda e, f: (0, 0)),
            pl.BlockSpec((1, _G), lambda e, f: (0, 0)),
            pl.BlockSpec((_D, _GE), lambda e, f: (0, 0)),
            pl.BlockSpec((1, _GE), lambda e, f: (0, 0)),
            pl.BlockSpec((_GE, _D), lambda e, f: (0, 0)),
            pl.BlockSpec((1, _D, _FC), lambda e, f: (e, 0, f)),
            pl.BlockSpec((1, 1, _FC), lambda e, f: (e, 0, f)),
            pl.BlockSpec((1, _FC, _D), lambda e, f: (e, f, 0)),
        ],
        out_specs=pl.BlockSpec((_T, _D), lambda e, f: (0, 0)),
        out_shape=jax.ShapeDtypeStruct((_T, _D), jnp.float32),
        scratch_shapes=[pltpu.VMEM((_T, _GE), jnp.float32)],
    )(x_bf, wg_bf, bg2, wr_bf, br2, b2r, w1r, b1r, w2r)
    return out


# speedup vs baseline: 1.0457x; 1.0088x over previous
import jax
import jax.numpy as jnp
from jax.experimental import pallas as pl
from jax.experimental.pallas import tpu as pltpu

_T, _D, _F, _G, _E = 2048, 768, 3072, 3, 8
_GE = _G * _E
_FC = 1536
_NF = _F // _FC


def _moe_body(x_ref, wg_ref, bg_ref, wr_ref, br_ref, b2r_ref,
              w1_ref, b1_ref, w2_ref, out_ref, w_scr):
    e = pl.program_id(0)
    f = pl.program_id(1)

    @pl.when((e == 0) & (f == 0))
    def _init():
        x = x_ref[...]
        gl = jnp.dot(x, wg_ref[...], preferred_element_type=jnp.float32)
        gl = gl + bg_ref[...]
        gl = gl - jnp.max(gl, axis=1, keepdims=True)
        gp = jnp.exp(gl)
        gp = gp / jnp.sum(gp, axis=1, keepdims=True)
        el = jnp.dot(x, wr_ref[...], preferred_element_type=jnp.float32)
        el = el + br_ref[...]
        cols = []
        for g in range(_G):
            sl = el[:, g * _E:(g + 1) * _E]
            sl = sl - jnp.max(sl, axis=1, keepdims=True)
            p = jnp.exp(sl)
            p = p / jnp.sum(p, axis=1, keepdims=True)
            cols.append(p * gp[:, g:g + 1])
        w = jnp.concatenate(cols, axis=1)
        w_scr[...] = w * 0.5
        out_ref[...] = jnp.dot(w, b2r_ref[...],
                               preferred_element_type=jnp.float32)

    x = x_ref[...]
    w1 = w1_ref[0].astype(jnp.bfloat16)
    t = jnp.dot(x, w1, preferred_element_type=jnp.float32)
    t = t + b1_ref[0]
    lane = jax.lax.broadcasted_iota(jnp.int32, (_T, _GE), 1)
    wselh = jnp.sum(jnp.where(lane == e, w_scr[...], 0.0),
                    axis=1, keepdims=True).astype(jnp.bfloat16)
    t_bf = t.astype(jnp.bfloat16)
    v = jax.lax.erf(t_bf * jnp.bfloat16(0.7071067811865476))
    a = t_bf * wselh
    h = a + a * v
    w2 = w2_ref[0].astype(jnp.bfloat16)
    out_ref[...] += jnp.dot(h, w2, preferred_element_type=jnp.float32)


def kernel(h_fused, Wg, bg, Wr, br, W1, b1, W2, b2):
    x_bf = h_fused.astype(jnp.bfloat16)
    wg_bf = Wg.astype(jnp.bfloat16)
    wr_bf = Wr.transpose(1, 0, 2).reshape(_D, _GE).astype(jnp.bfloat16)
    bg2 = bg.reshape(1, _G)
    br2 = br.reshape(1, _GE)
    w1r = W1.reshape(_GE, _D, _F)
    b1r = b1.reshape(_GE, 1, _F)
    w2r = W2.reshape(_GE, _F, _D)
    b2r = b2.reshape(_GE, _D)

    out = pl.pallas_call(
        _moe_body,
        grid=(_GE, _NF),
        in_specs=[
            pl.BlockSpec((_T, _D), lambda e, f: (0, 0)),
            pl.BlockSpec((_D, _G), lambda e, f: (0, 0)),
            pl.BlockSpec((1, _G), lambda e, f: (0, 0)),
            pl.BlockSpec((_D, _GE), lambda e, f: (0, 0)),
            pl.BlockSpec((1, _GE), lambda e, f: (0, 0)),
            pl.BlockSpec((_GE, _D), lambda e, f: (0, 0)),
            pl.BlockSpec((1, _D, _FC), lambda e, f: (e, 0, f)),
            pl.BlockSpec((1, 1, _FC), lambda e, f: (e, 0, f)),
            pl.BlockSpec((1, _FC, _D), lambda e, f: (e, f, 0)),
        ],
        out_specs=pl.BlockSpec((_T, _D), lambda e, f: (0, 0)),
        out_shape=jax.ShapeDtypeStruct((_T, _D), jnp.float32),
        scratch_shapes=[pltpu.VMEM((_T, _GE), jnp.float32)],
    )(x_bf, wg_bf, bg2, wr_bf, br2, b2r, w1r, b1r, w2r)
    return out


# speedup vs baseline: 1.0629x; 1.0164x over previous
import jax
import jax.numpy as jnp
from jax.experimental import pallas as pl
from jax.experimental.pallas import tpu as pltpu

_T, _D, _F, _G, _E = 2048, 768, 3072, 3, 8
_GE = _G * _E
_FC = 1536
_NF = _F // _FC


def _moe_body(x_ref, wg_ref, bg_ref, wr_ref, br_ref, b2r_ref,
              w1_ref, b1_ref, w2_ref, out_ref, w_scr):
    e = pl.program_id(0)
    f = pl.program_id(1)

    @pl.when((e == 0) & (f == 0))
    def _init():
        x = x_ref[...]
        gl = jnp.dot(x, wg_ref[...], preferred_element_type=jnp.float32)
        gl = gl + bg_ref[...]
        gl = gl - jnp.max(gl, axis=1, keepdims=True)
        gp = jnp.exp(gl)
        gp = gp / jnp.sum(gp, axis=1, keepdims=True)
        el = jnp.dot(x, wr_ref[...], preferred_element_type=jnp.float32)
        el = el + br_ref[...]
        cols = []
        for g in range(_G):
            sl = el[:, g * _E:(g + 1) * _E]
            sl = sl - jnp.max(sl, axis=1, keepdims=True)
            p = jnp.exp(sl)
            p = p / jnp.sum(p, axis=1, keepdims=True)
            cols.append(p * gp[:, g:g + 1])
        w = jnp.concatenate(cols, axis=1)
        w_scr[...] = w * 0.5
        out_ref[...] = jnp.dot(w, b2r_ref[...],
                               preferred_element_type=jnp.float32)

    w1 = w1_ref[0].astype(jnp.bfloat16)
    w2 = w2_ref[0].astype(jnp.bfloat16)
    b1c = b1_ref[0].astype(jnp.bfloat16)
    lane = jax.lax.broadcasted_iota(jnp.int32, (_T, _GE), 1)
    wselh = jnp.sum(jnp.where(lane == e, w_scr[...], 0.0),
                    axis=1, keepdims=True).astype(jnp.bfloat16)
    _H = _T // 2
    for m in range(2):
        sl = slice(m * _H, (m + 1) * _H)
        x = x_ref[sl, :]
        t = jnp.dot(x, w1, preferred_element_type=jnp.float32)
        t_bf = t.astype(jnp.bfloat16) + b1c
        v = jax.lax.erf(t_bf * jnp.bfloat16(0.7071067811865476))
        a = t_bf * wselh[sl, :]
        h = a + a * v
        out_ref[sl, :] += jnp.dot(h, w2,
                                  preferred_element_type=jnp.float32)


def kernel(h_fused, Wg, bg, Wr, br, W1, b1, W2, b2):
    x_bf = h_fused.astype(jnp.bfloat16)
    wg_bf = Wg.astype(jnp.bfloat16)
    wr_bf = Wr.transpose(1, 0, 2).reshape(_D, _GE).astype(jnp.bfloat16)
    bg2 = bg.reshape(1, _G)
    br2 = br.reshape(1, _GE)
    w1r = W1.reshape(_GE, _D, _F)
    b1r = b1.reshape(_GE, 1, _F)
    w2r = W2.reshape(_GE, _F, _D)
    b2r = b2.reshape(_GE, _D)

    out = pl.pallas_call(
        _moe_body,
        grid=(_GE, _NF),
        in_specs=[
            pl.BlockSpec((_T, _D), lambda e, f: (0, 0)),
            pl.BlockSpec((_D, _G), lambda e, f: (0, 0)),
            pl.BlockSpec((1, _G), lambda e, f: (0, 0)),
            pl.BlockSpec((_D, _GE), lambda e, f: (0, 0)),
            pl.BlockSpec((1, _GE), lambda e, f: (0, 0)),
            pl.BlockSpec((_GE, _D), lambda e, f: (0, 0)),
            pl.BlockSpec((1, _D, _FC), lambda e, f: (e, 0, f)),
            pl.BlockSpec((1, 1, _FC), lambda e, f: (e, 0, f)),
            pl.BlockSpec((1, _FC, _D), lambda e, f: (e, f, 0)),
        ],
        out_specs=pl.BlockSpec((_T, _D), lambda e, f: (0, 0)),
        out_shape=jax.ShapeDtypeStruct((_T, _D), jnp.float32),
        scratch_shapes=[pltpu.VMEM((_T, _GE), jnp.float32)],
    )(x_bf, wg_bf, bg2, wr_bf, br2, b2r, w1r, b1r, w2r)
    return out


# speedup vs baseline: 1.0645x; 1.0015x over previous
import jax
import jax.numpy as jnp
from jax.experimental import pallas as pl
from jax.experimental.pallas import tpu as pltpu

_T, _D, _F, _G, _E = 2048, 768, 3072, 3, 8
_GE = _G * _E
_FC = 1536
_NF = _F // _FC


def _moe_body(x_ref, wg_ref, bg_ref, wr_ref, br_ref, b2r_ref,
              w1_ref, b1_ref, w2_ref, out_ref, w_scr):
    e = pl.program_id(0)
    f = pl.program_id(1)

    @pl.when((e == 0) & (f == 0))
    def _init():
        x = x_ref[...]
        gl = jnp.dot(x, wg_ref[...], preferred_element_type=jnp.float32)
        gl = gl + bg_ref[...]
        gl = gl - jnp.max(gl, axis=1, keepdims=True)
        gp = jnp.exp(gl)
        gp = gp / jnp.sum(gp, axis=1, keepdims=True)
        el = jnp.dot(x, wr_ref[...], preferred_element_type=jnp.float32)
        el = el + br_ref[...]
        cols = []
        for g in range(_G):
            sl = el[:, g * _E:(g + 1) * _E]
            sl = sl - jnp.max(sl, axis=1, keepdims=True)
            p = jnp.exp(sl)
            p = p / jnp.sum(p, axis=1, keepdims=True)
            cols.append(p * gp[:, g:g + 1])
        w = jnp.concatenate(cols, axis=1)
        w_scr[...] = w * 0.5
        out_ref[...] = jnp.dot(w, b2r_ref[...],
                               preferred_element_type=jnp.float32)

    w1 = w1_ref[0].astype(jnp.bfloat16)
    w2 = w2_ref[0].astype(jnp.bfloat16)
    b1c = b1_ref[0].astype(jnp.bfloat16)
    lane = jax.lax.broadcasted_iota(jnp.int32, (_T, _GE), 1)
    wselh = jnp.sum(jnp.where(lane == e, w_scr[...], 0.0),
                    axis=1, keepdims=True).astype(jnp.bfloat16)
    _H = _T // 4
    for m in range(4):
        sl = slice(m * _H, (m + 1) * _H)
        x = x_ref[sl, :]
        t = jnp.dot(x, w1, preferred_element_type=jnp.float32)
        t_bf = t.astype(jnp.bfloat16) + b1c
        v = jax.lax.erf(t_bf * jnp.bfloat16(0.7071067811865476))
        a = t_bf * wselh[sl, :]
        h = a + a * v
        out_ref[sl, :] += jnp.dot(h, w2,
                                  preferred_element_type=jnp.float32)


def kernel(h_fused, Wg, bg, Wr, br, W1, b1, W2, b2):
    x_bf = h_fused.astype(jnp.bfloat16)
    wg_bf = Wg.astype(jnp.bfloat16)
    wr_bf = Wr.transpose(1, 0, 2).reshape(_D, _GE).astype(jnp.bfloat16)
    bg2 = bg.reshape(1, _G)
    br2 = br.reshape(1, _GE)
    w1r = W1.reshape(_GE, _D, _F)
    b1r = b1.reshape(_GE, 1, _F)
    w2r = W2.reshape(_GE, _F, _D)
    b2r = b2.reshape(_GE, _D)

    out = pl.pallas_call(
        _moe_body,
        grid=(_GE, _NF),
        in_specs=[
            pl.BlockSpec((_T, _D), lambda e, f: (0, 0)),
            pl.BlockSpec((_D, _G), lambda e, f: (0, 0)),
            pl.BlockSpec((1, _G), lambda e, f: (0, 0)),
            pl.BlockSpec((_D, _GE), lambda e, f: (0, 0)),
            pl.BlockSpec((1, _GE), lambda e, f: (0, 0)),
            pl.BlockSpec((_GE, _D), lambda e, f: (0, 0)),
            pl.BlockSpec((1, _D, _FC), lambda e, f: (e, 0, f)),
            pl.BlockSpec((1, 1, _FC), lambda e, f: (e, 0, f)),
            pl.BlockSpec((1, _FC, _D), lambda e, f: (e, f, 0)),
        ],
        out_specs=pl.BlockSpec((_T, _D), lambda e, f: (0, 0)),
        out_shape=jax.ShapeDtypeStruct((_T, _D), jnp.float32),
        scratch_shapes=[pltpu.VMEM((_T, _GE), jnp.float32)],
    )(x_bf, wg_bf, bg2, wr_bf, br2, b2r, w1r, b1r, w2r)
    return out


# speedup vs baseline: 1.0777x; 1.0124x over previous
import jax
import jax.numpy as jnp
from jax.experimental import pallas as pl
from jax.experimental.pallas import tpu as pltpu

_T, _D, _F, _G, _E = 2048, 768, 3072, 3, 8
_GE = _G * _E
_FC = 1536
_NF = _F // _FC
_NM = 4
_H = _T // _NM


def _moe_body(x_ref, wg_ref, bg_ref, wr_ref, br_ref, b2r_ref,
              w1_ref, b1_ref, w2_ref, out_ref, w_scr):
    e = pl.program_id(0)

    @pl.when(e == 0)
    def _init():
        x = x_ref[...]
        gl = jnp.dot(x, wg_ref[...], preferred_element_type=jnp.float32)
        gl = gl + bg_ref[...]
        gl = gl - jnp.max(gl, axis=1, keepdims=True)
        gp = jnp.exp(gl)
        gp = gp / jnp.sum(gp, axis=1, keepdims=True)
        el = jnp.dot(x, wr_ref[...], preferred_element_type=jnp.float32)
        el = el + br_ref[...]
        cols = []
        for g in range(_G):
            sl = el[:, g * _E:(g + 1) * _E]
            sl = sl - jnp.max(sl, axis=1, keepdims=True)
            p = jnp.exp(sl)
            p = p / jnp.sum(p, axis=1, keepdims=True)
            cols.append(p * gp[:, g:g + 1])
        w = jnp.concatenate(cols, axis=1)
        w_scr[...] = (w * 0.5).astype(jnp.bfloat16)
        out_ref[...] = jnp.dot(w, b2r_ref[...],
                               preferred_element_type=jnp.float32)

    lane = jax.lax.broadcasted_iota(jnp.int32, (_T, _GE), 1)
    wselh = jnp.sum(jnp.where(lane == e, w_scr[...], jnp.bfloat16(0.0)),
                    axis=1, keepdims=True)
    for f in range(_NF):
        fs = slice(f * _FC, (f + 1) * _FC)
        w1 = w1_ref[0, :, fs].astype(jnp.bfloat16)
        w2 = w2_ref[0, fs, :].astype(jnp.bfloat16)
        b1c = b1_ref[0, :, fs].astype(jnp.bfloat16)
        for m in range(_NM):
            sl = slice(m * _H, (m + 1) * _H)
            x = x_ref[sl, :]
            t = jnp.dot(x, w1, preferred_element_type=jnp.float32)
            t_bf = t.astype(jnp.bfloat16) + b1c
            v = jax.lax.erf(t_bf * jnp.bfloat16(0.7071067811865476))
            a = t_bf * wselh[sl, :]
            h = a + a * v
            out_ref[sl, :] += jnp.dot(h, w2,
                                      preferred_element_type=jnp.float32)


def kernel(h_fused, Wg, bg, Wr, br, W1, b1, W2, b2):
    x_bf = h_fused.astype(jnp.bfloat16)
    wg_bf = Wg.astype(jnp.bfloat16)
    wr_bf = Wr.transpose(1, 0, 2).reshape(_D, _GE).astype(jnp.bfloat16)
    bg2 = bg.reshape(1, _G)
    br2 = br.reshape(1, _GE)
    w1r = W1.reshape(_GE, _D, _F)
    b1r = b1.reshape(_GE, 1, _F)
    w2r = W2.reshape(_GE, _F, _D)
    b2r = b2.reshape(_GE, _D)

    out = pl.pallas_call(
        _moe_body,
        grid=(_GE,),
        in_specs=[
            pl.BlockSpec((_T, _D), lambda e: (0, 0)),
            pl.BlockSpec((_D, _G), lambda e: (0, 0)),
            pl.BlockSpec((1, _G), lambda e: (0, 0)),
            pl.BlockSpec((_D, _GE), lambda e: (0, 0)),
            pl.BlockSpec((1, _GE), lambda e: (0, 0)),
            pl.BlockSpec((_GE, _D), lambda e: (0, 0)),
            pl.BlockSpec((1, _D, _F), lambda e: (e, 0, 0)),
            pl.BlockSpec((1, 1, _F), lambda e: (e, 0, 0)),
            pl.BlockSpec((1, _F, _D), lambda e: (e, 0, 0)),
        ],
        out_specs=pl.BlockSpec((_T, _D), lambda e: (0, 0)),
        out_shape=jax.ShapeDtypeStruct((_T, _D), jnp.float32),
        scratch_shapes=[pltpu.VMEM((_T, _GE), jnp.bfloat16)],
        compiler_params=pltpu.CompilerParams(
            vmem_limit_bytes=67108864),
    )(x_bf, wg_bf, bg2, wr_bf, br2, b2r, w1r, b1r, w2r)
    return out


# speedup vs baseline: 1.0791x; 1.0013x over previous
import jax
import jax.numpy as jnp
from jax.experimental import pallas as pl
from jax.experimental.pallas import tpu as pltpu

_T, _D, _F, _G, _E = 2048, 768, 3072, 3, 8
_GE = _G * _E
_FC = 1536
_NF = _F // _FC
_NM = 2
_H = _T // _NM


def _moe_body(x_ref, wg_ref, bg_ref, wr_ref, br_ref, b2r_ref,
              w1_ref, b1_ref, w2_ref, out_ref, w_scr):
    e = pl.program_id(0)

    @pl.when(e == 0)
    def _init():
        x = x_ref[...]
        gl = jnp.dot(x, wg_ref[...], preferred_element_type=jnp.float32)
        gl = gl + bg_ref[...]
        gl = gl - jnp.max(gl, axis=1, keepdims=True)
        gp = jnp.exp(gl)
        gp = gp / jnp.sum(gp, axis=1, keepdims=True)
        el = jnp.dot(x, wr_ref[...], preferred_element_type=jnp.float32)
        el = el + br_ref[...]
        cols = []
        for g in range(_G):
            sl = el[:, g * _E:(g + 1) * _E]
            sl = sl - jnp.max(sl, axis=1, keepdims=True)
            p = jnp.exp(sl)
            p = p / jnp.sum(p, axis=1, keepdims=True)
            cols.append(p * gp[:, g:g + 1])
        w = jnp.concatenate(cols, axis=1)
        w_scr[...] = (w * 0.5).astype(jnp.bfloat16)
        out_ref[...] = jnp.dot(w, b2r_ref[...],
                               preferred_element_type=jnp.float32)

    lane = jax.lax.broadcasted_iota(jnp.int32, (_T, _GE), 1)
    wselh = jnp.sum(jnp.where(lane == e, w_scr[...], jnp.bfloat16(0.0)),
                    axis=1, keepdims=True)
    for f in range(_NF):
        fs = slice(f * _FC, (f + 1) * _FC)
        w1 = w1_ref[0, :, fs].astype(jnp.bfloat16)
        w2 = w2_ref[0, fs, :].astype(jnp.bfloat16)
        b1c = b1_ref[0, :, fs].astype(jnp.bfloat16)
        for m in range(_NM):
            sl = slice(m * _H, (m + 1) * _H)
            x = x_ref[sl, :]
            t = jnp.dot(x, w1, preferred_element_type=jnp.float32)
            t_bf = t.astype(jnp.bfloat16) + b1c
            v = jax.lax.erf(t_bf * jnp.bfloat16(0.7071067811865476))
            a = t_bf * wselh[sl, :]
            h = a + a * v
            out_ref[sl, :] += jnp.dot(h, w2,
                                      preferred_element_type=jnp.float32)


def kernel(h_fused, Wg, bg, Wr, br, W1, b1, W2, b2):
    x_bf = h_fused.astype(jnp.bfloat16)
    wg_bf = Wg.astype(jnp.bfloat16)
    wr_bf = Wr.transpose(1, 0, 2).reshape(_D, _GE).astype(jnp.bfloat16)
    bg2 = bg.reshape(1, _G)
    br2 = br.reshape(1, _GE)
    w1r = W1.reshape(_GE, _D, _F)
    b1r = b1.reshape(_GE, 1, _F)
    w2r = W2.reshape(_GE, _F, _D)
    b2r = b2.reshape(_GE, _D)

    out = pl.pallas_call(
        _moe_body,
        grid=(_GE,),
        in_specs=[
            pl.BlockSpec((_T, _D), lambda e: (0, 0)),
            pl.BlockSpec((_D, _G), lambda e: (0, 0)),
            pl.BlockSpec((1, _G), lambda e: (0, 0)),
            pl.BlockSpec((_D, _GE), lambda e: (0, 0)),
            pl.BlockSpec((1, _GE), lambda e: (0, 0)),
            pl.BlockSpec((_GE, _D), lambda e: (0, 0)),
            pl.BlockSpec((1, _D, _F), lambda e: (e, 0, 0)),
            pl.BlockSpec((1, 1, _F), lambda e: (e, 0, 0)),
            pl.BlockSpec((1, _F, _D), lambda e: (e, 0, 0)),
        ],
        out_specs=pl.BlockSpec((_T, _D), lambda e: (0, 0)),
        out_shape=jax.ShapeDtypeStruct((_T, _D), jnp.float32),
        scratch_shapes=[pltpu.VMEM((_T, _GE), jnp.bfloat16)],
        compiler_params=pltpu.CompilerParams(
            vmem_limit_bytes=67108864),
    )(x_bf, wg_bf, bg2, wr_bf, br2, b2r, w1r, b1r, w2r)
    return out
